# baseline (speedup 1.0000x reference)
import dataclasses
import functools

import numpy as np
import jax
import jax.numpy as jnp
from jax import lax
from jax.experimental import pallas as pl
from jax.experimental.pallas import tpu as pltpu
from jax.experimental.pallas import tpu_sc as plsc

_NUM_FIELDS = 26
_EMBED_DIM = 16
_EMBED_OUT = _NUM_FIELDS * _EMBED_DIM
_B = 16384
_VOCAB = 100000 * _NUM_FIELDS
_N_E = _EMBED_OUT * _B
_N_L = _NUM_FIELDS * _B
_OFFS = np.arange(_NUM_FIELDS, dtype=np.int32) * 100000

_NC = 2
_NS = 16
_NW = _NC * _NS
_EPW = _N_E // _NW
_LPW = _N_L // _NW
_CHUNK = 4096
_LCHUNK = 3328

_S_MAT = np.tile(np.eye(_EMBED_DIM, dtype=np.float32), (_NUM_FIELDS, 1))


def _sc_compiler_params():
    cp = pltpu.CompilerParams(use_tc_tiling_on_sc=False)
    if "needs_layout_passes" in pltpu.CompilerParams.__dataclass_fields__:
        cp = dataclasses.replace(cp, needs_layout_passes=False)
    return cp


@functools.lru_cache(maxsize=1)
def _build_sc_gather():
    @functools.partial(
        pl.kernel,
        out_type=[
            jax.ShapeDtypeStruct((_N_E,), jnp.float32),
            jax.ShapeDtypeStruct((_N_L,), jnp.float32),
        ],
        mesh=plsc.VectorSubcoreMesh(core_axis_name="c", subcore_axis_name="s"),
        scratch_types=[
            pltpu.VMEM((_CHUNK,), jnp.int32),
            pltpu.VMEM((_CHUNK,), jnp.float32),
            pltpu.VMEM((_LCHUNK,), jnp.int32),
            pltpu.VMEM((_LCHUNK,), jnp.float32),
            pltpu.SemaphoreType.DMA,
        ],
        compiler_params=_sc_compiler_params(),
    )
    def _sc_gather(eidx_hbm, lidx_hbm, et_flat, l_flat,
                   gt_out, lval_out,
                   eidx_v, eval_v, lidx_v, lval_v, sem):
        wid = lax.axis_index("s") * _NC + lax.axis_index("c")
        ebase = wid * _EPW
        lbase = wid * _LPW

        @pl.loop(0, _EPW, step=_CHUNK)
        def _(off):
            start = ebase + off
            pltpu.sync_copy(eidx_hbm.at[pl.ds(start, _CHUNK)], eidx_v)
            pltpu.async_copy(et_flat.at[eidx_v], eval_v, sem).wait()
            pltpu.sync_copy(eval_v, gt_out.at[pl.ds(start, _CHUNK)])

        @pl.loop(0, _LPW, step=_LCHUNK)
        def _(off):
            start = lbase + off
            pltpu.sync_copy(lidx_hbm.at[pl.ds(start, _LCHUNK)], lidx_v)
            pltpu.async_copy(l_flat.at[lidx_v], lval_v, sem).wait()
            pltpu.sync_copy(lval_v, lval_out.at[pl.ds(start, _LCHUNK)])

    return _sc_gather


_VMAIN = _VOCAB - 64
_TBASE = _EMBED_DIM * _VMAIN


def _compact_body(et_ref, tail_ref, out_ref, sem):
    copies = [
        pltpu.make_async_copy(
            et_ref.at[d, pl.ds(0, _VMAIN)],
            out_ref.at[pl.ds(d * _VMAIN, _VMAIN)],
            sem,
        )
        for d in range(_EMBED_DIM)
    ]
    copies.append(
        pltpu.make_async_copy(tail_ref, out_ref.at[pl.ds(_TBASE, 1024)], sem)
    )
    for c in copies:
        c.start()
    for c in copies:
        c.wait()


def _compact_et(ET, tail_flat):
    return pl.pallas_call(
        _compact_body,
        in_specs=[pl.BlockSpec(memory_space=pl.ANY),
                  pl.BlockSpec(memory_space=pl.ANY)],
        out_specs=pl.BlockSpec(memory_space=pl.ANY),
        out_shape=jax.ShapeDtypeStruct((_EMBED_DIM * _VOCAB,), jnp.float32),
        scratch_shapes=[pltpu.SemaphoreType.DMA],
    )(ET, tail_flat)


_BLK = 2048
_NB = _B // _BLK
_CONTRACT0 = (((0,), (0,)), ((), ()))


def _tc_body(gt_ref, lval_ref, W1_ref, b1_ref, g1_ref, be1_ref,
             W2_ref, b2_ref, g2_ref, be2_ref, W3_ref, sc_ref, S_ref,
             out_ref, h1_s, base_s):
    i = pl.program_id(0)
    M = gt_ref[...]
    h1 = lax.dot_general(M, W1_ref[...], _CONTRACT0,
                         preferred_element_type=jnp.float32) + b1_ref[...]
    h1_s[pl.ds(i * _BLK, _BLK), :] = h1

    s = lax.dot_general(M, S_ref[...], _CONTRACT0,
                        preferred_element_type=jnp.float32)
    fm = 0.5 * (jnp.sum(s * s, axis=1) - jnp.sum(M * M, axis=0))
    lin = jnp.sum(lval_ref[...], axis=0)
    base_s[pl.ds(i * _BLK, _BLK)] = lin + fm + sc_ref[0]

    @pl.when(i == _NB - 1)
    def _():
        H1 = h1_s[...]
        mu1 = jnp.mean(H1, axis=0, keepdims=True)
        d1 = H1 - mu1
        var1 = jnp.mean(d1 * d1, axis=0, keepdims=True)
        a1 = g1_ref[...] * lax.rsqrt(var1 + 1e-5)
        N1 = jnp.maximum(d1 * a1 + be1_ref[...], 0.0)
        H2 = jnp.dot(N1, W2_ref[...], preferred_element_type=jnp.float32) + b2_ref[...]
        mu2 = jnp.mean(H2, axis=0, keepdims=True)
        d2 = H2 - mu2
        var2 = jnp.mean(d2 * d2, axis=0, keepdims=True)
        a2 = g2_ref[...] * lax.rsqrt(var2 + 1e-5)
        N2 = jnp.maximum(d2 * a2 + be2_ref[...], 0.0)
        mlp = jnp.dot(N2, W3_ref[...], preferred_element_type=jnp.float32)[:, 0]
        z = base_s[...] + mlp
        e = jnp.exp(-jnp.abs(z))
        out_ref[...] = jnp.where(z >= 0, 1.0 / (1.0 + e), e / (1.0 + e))


def _tc_mlp(GT, lvalT, W1, b1, g1, be1, W2, b2, g2, be2, W3, sc):
    full = lambda shape: pl.BlockSpec(shape, lambda i: tuple(0 for _ in shape))
    return pl.pallas_call(
        _tc_body,
        grid=(_NB,),
        in_specs=[
            pl.BlockSpec((_EMBED_OUT, _BLK), lambda i: (0, i)),
            pl.BlockSpec((_NUM_FIELDS, _BLK), lambda i: (0, i)),
            full((_EMBED_OUT, 128)),
            full((1, 128)),
            full((1, 128)),
            full((1, 128)),
            full((128, 128)),
            full((1, 128)),
            full((1, 128)),
            full((1, 128)),
            full((128, 1)),
            pl.BlockSpec(memory_space=pltpu.SMEM),
            full((_EMBED_OUT, _EMBED_DIM)),
        ],
        out_specs=pl.BlockSpec((_B,), lambda i: (0,)),
        out_shape=jax.ShapeDtypeStruct((_B,), jnp.float32),
        scratch_shapes=[
            pltpu.VMEM((_B, 128), jnp.float32),
            pltpu.VMEM((_B,), jnp.float32),
        ],
    )(GT, lvalT, W1, b1.reshape(1, 128), g1.reshape(1, 128), be1.reshape(1, 128),
      W2, b2.reshape(1, 128), g2.reshape(1, 128), be2.reshape(1, 128),
      W3, sc, jnp.asarray(_S_MAT))


def kernel(x, E, L, bias, W1, b1, g1, be1, W2, b2, g2, be2, W3, b3):
    idxT = x.T.astype(jnp.int32) + jnp.asarray(_OFFS)[:, None]
    d_off = jnp.arange(_EMBED_DIM, dtype=jnp.int32)[None, :, None]
    v = idxT[:, None, :]
    eidx = jnp.where(v < _VMAIN,
                     d_off * _VMAIN + v,
                     _TBASE + d_off * 64 + (v - _VMAIN))
    eidx = eidx.reshape(_N_E)
    lidx = idxT.reshape(_N_L)
    tail_flat = lax.slice(E, (_VMAIN, 0), (_VOCAB, _EMBED_DIM)).T.reshape(1024)
    et_flat = _compact_et(E.T, tail_flat)
    l_flat = L.reshape(_VOCAB)
    gt_flat, lval_flat = _build_sc_gather()(eidx, lidx, et_flat, l_flat)
    GT = gt_flat.reshape(_EMBED_OUT, _B)
    lvalT = lval_flat.reshape(_NUM_FIELDS, _B)
    sc = (bias + b3).reshape(1)
    return _tc_mlp(GT, lvalT, W1, b1, g1, be1, W2, b2, g2, be2, W3, sc)


# speedup vs baseline: 7.7756x; 7.7756x over previous
import dataclasses
import functools

import numpy as np
import jax
import jax.numpy as jnp
from jax import lax
from jax.experimental import pallas as pl
from jax.experimental.pallas import tpu as pltpu
from jax.experimental.pallas import tpu_sc as plsc

_NUM_FIELDS = 26
_EMBED_DIM = 16
_EMBED_OUT = _NUM_FIELDS * _EMBED_DIM
_B = 16384
_VOCAB = 100000 * _NUM_FIELDS
_N_E = _EMBED_OUT * _B
_N_L = _NUM_FIELDS * _B
_OFFS = np.arange(_NUM_FIELDS, dtype=np.int32) * 100000

_NC = 2
_NS = 16
_NW = _NC * _NS
_EPW = _N_E // _NW
_LPW = _N_L // _NW
_CHUNK = 4096
_LCHUNK = 3328

_S_MAT = np.tile(np.eye(_EMBED_DIM, dtype=np.float32), (_NUM_FIELDS, 1))


def _sc_compiler_params():
    cp = pltpu.CompilerParams(use_tc_tiling_on_sc=False)
    if "needs_layout_passes" in pltpu.CompilerParams.__dataclass_fields__:
        cp = dataclasses.replace(cp, needs_layout_passes=False)
    return cp


@functools.lru_cache(maxsize=1)
def _build_sc_gather():
    @functools.partial(
        pl.kernel,
        out_type=[
            jax.ShapeDtypeStruct((_N_E,), jnp.float32),
            jax.ShapeDtypeStruct((_N_L,), jnp.float32),
        ],
        mesh=plsc.VectorSubcoreMesh(core_axis_name="c", subcore_axis_name="s"),
        scratch_types=[
            pltpu.VMEM((_CHUNK,), jnp.int32),
            pltpu.VMEM((_CHUNK,), jnp.float32),
            pltpu.VMEM((_LCHUNK,), jnp.int32),
            pltpu.VMEM((_LCHUNK,), jnp.float32),
            pltpu.SemaphoreType.DMA,
        ],
        compiler_params=_sc_compiler_params(),
    )
    def _sc_gather(eidx_hbm, lidx_hbm, et_flat, l_flat,
                   gt_out, lval_out,
                   eidx_v, eval_v, lidx_v, lval_v, sem):
        wid = lax.axis_index("s") * _NC + lax.axis_index("c")
        ebase = wid * _EPW
        lbase = wid * _LPW

        @pl.loop(0, _EPW, step=_CHUNK)
        def _(off):
            start = ebase + off
            pltpu.sync_copy(eidx_hbm.at[pl.ds(start, _CHUNK)], eidx_v)
            pltpu.async_copy(et_flat.at[eidx_v], eval_v, sem).wait()
            pltpu.sync_copy(eval_v, gt_out.at[pl.ds(start, _CHUNK)])

        @pl.loop(0, _LPW, step=_LCHUNK)
        def _(off):
            start = lbase + off
            pltpu.sync_copy(lidx_hbm.at[pl.ds(start, _LCHUNK)], lidx_v)
            pltpu.async_copy(l_flat.at[lidx_v], lval_v, sem).wait()
            pltpu.sync_copy(lval_v, lval_out.at[pl.ds(start, _LCHUNK)])

    return _sc_gather


_VMAIN = _VOCAB - 64
_TBASE = _EMBED_DIM * _VMAIN


_CBLK = 262144
_CGRID = 10
_CLAST = _VMAIN - (_CGRID - 1) * _CBLK


def _compact_body(et_ref, tail_ref, out_ref, sem):
    i = pl.program_id(0)

    def emit(size):
        copies = [
            pltpu.make_async_copy(
                et_ref.at[d, pl.ds(0, size)],
                out_ref.at[pl.ds(d * _VMAIN + i * _CBLK, size)],
                sem,
            )
            for d in range(_EMBED_DIM)
        ]
        for c in copies:
            c.start()
        for c in copies:
            c.wait()

    @pl.when(i < _CGRID - 1)
    def _():
        emit(_CBLK)

    @pl.when(i == _CGRID - 1)
    def _():
        emit(_CLAST)

    @pl.when(i == 0)
    def _():
        c = pltpu.make_async_copy(tail_ref, out_ref.at[pl.ds(_TBASE, 1024)], sem)
        c.start()
        c.wait()


def _compact_et(ET, tail_flat):
    return pl.pallas_call(
        _compact_body,
        grid=(_CGRID,),
        in_specs=[pl.BlockSpec((_EMBED_DIM, _CBLK), lambda i: (0, i)),
                  pl.BlockSpec(memory_space=pl.ANY)],
        out_specs=pl.BlockSpec(memory_space=pl.ANY),
        out_shape=jax.ShapeDtypeStruct((_EMBED_DIM * _VOCAB,), jnp.float32),
        scratch_shapes=[pltpu.SemaphoreType.DMA],
    )(ET, tail_flat)


_BLK = 2048
_NB = _B // _BLK
_CONTRACT0 = (((0,), (0,)), ((), ()))


def _tc_body(gt_ref, lval_ref, W1_ref, b1_ref, g1_ref, be1_ref,
             W2_ref, b2_ref, g2_ref, be2_ref, W3_ref, sc_ref, S_ref,
             out_ref, h1_s, base_s):
    i = pl.program_id(0)
    M = gt_ref[...]
    h1 = lax.dot_general(M, W1_ref[...], _CONTRACT0,
                         preferred_element_type=jnp.float32) + b1_ref[...]
    h1_s[pl.ds(i * _BLK, _BLK), :] = h1

    s = lax.dot_general(M, S_ref[...], _CONTRACT0,
                        preferred_element_type=jnp.float32)
    fm = 0.5 * (jnp.sum(s * s, axis=1) - jnp.sum(M * M, axis=0))
    lin = jnp.sum(lval_ref[...], axis=0)
    base_s[pl.ds(i * _BLK, _BLK)] = lin + fm + sc_ref[0]

    @pl.when(i == _NB - 1)
    def _():
        H1 = h1_s[...]
        mu1 = jnp.mean(H1, axis=0, keepdims=True)
        d1 = H1 - mu1
        var1 = jnp.mean(d1 * d1, axis=0, keepdims=True)
        a1 = g1_ref[...] * lax.rsqrt(var1 + 1e-5)
        N1 = jnp.maximum(d1 * a1 + be1_ref[...], 0.0)
        H2 = jnp.dot(N1, W2_ref[...], preferred_element_type=jnp.float32) + b2_ref[...]
        mu2 = jnp.mean(H2, axis=0, keepdims=True)
        d2 = H2 - mu2
        var2 = jnp.mean(d2 * d2, axis=0, keepdims=True)
        a2 = g2_ref[...] * lax.rsqrt(var2 + 1e-5)
        N2 = jnp.maximum(d2 * a2 + be2_ref[...], 0.0)
        mlp = jnp.dot(N2, W3_ref[...], preferred_element_type=jnp.float32)[:, 0]
        z = base_s[...] + mlp
        e = jnp.exp(-jnp.abs(z))
        out_ref[...] = jnp.where(z >= 0, 1.0 / (1.0 + e), e / (1.0 + e))


def _tc_mlp(GT, lvalT, W1, b1, g1, be1, W2, b2, g2, be2, W3, sc):
    full = lambda shape: pl.BlockSpec(shape, lambda i: tuple(0 for _ in shape))
    return pl.pallas_call(
        _tc_body,
        grid=(_NB,),
        in_specs=[
            pl.BlockSpec((_EMBED_OUT, _BLK), lambda i: (0, i)),
            pl.BlockSpec((_NUM_FIELDS, _BLK), lambda i: (0, i)),
            full((_EMBED_OUT, 128)),
            full((1, 128)),
            full((1, 128)),
            full((1, 128)),
            full((128, 128)),
            full((1, 128)),
            full((1, 128)),
            full((1, 128)),
            full((128, 1)),
            pl.BlockSpec(memory_space=pltpu.SMEM),
            full((_EMBED_OUT, _EMBED_DIM)),
        ],
        out_specs=pl.BlockSpec((_B,), lambda i: (0,)),
        out_shape=jax.ShapeDtypeStruct((_B,), jnp.float32),
        scratch_shapes=[
            pltpu.VMEM((_B, 128), jnp.float32),
            pltpu.VMEM((_B,), jnp.float32),
        ],
    )(GT, lvalT, W1, b1.reshape(1, 128), g1.reshape(1, 128), be1.reshape(1, 128),
      W2, b2.reshape(1, 128), g2.reshape(1, 128), be2.reshape(1, 128),
      W3, sc, jnp.asarray(_S_MAT))


def kernel(x, E, L, bias, W1, b1, g1, be1, W2, b2, g2, be2, W3, b3):
    idxT = x.T.astype(jnp.int32) + jnp.asarray(_OFFS)[:, None]
    d_off = jnp.arange(_EMBED_DIM, dtype=jnp.int32)[None, :, None]
    v = idxT[:, None, :]
    eidx = jnp.where(v < _VMAIN,
                     d_off * _VMAIN + v,
                     _TBASE + d_off * 64 + (v - _VMAIN))
    eidx = eidx.reshape(_N_E)
    lidx = idxT.reshape(_N_L)
    tail_flat = lax.slice(E, (_VMAIN, 0), (_VOCAB, _EMBED_DIM)).T.reshape(1024)
    et_flat = _compact_et(E.T, tail_flat)
    l_flat = L.reshape(_VOCAB)
    gt_flat, lval_flat = _build_sc_gather()(eidx, lidx, et_flat, l_flat)
    GT = gt_flat.reshape(_EMBED_OUT, _B)
    lvalT = lval_flat.reshape(_NUM_FIELDS, _B)
    sc = (bias + b3).reshape(1)
    return _tc_mlp(GT, lvalT, W1, b1, g1, be1, W2, b2, g2, be2, W3, sc)


# speedup vs baseline: 8.9806x; 1.1550x over previous
import dataclasses
import functools

import numpy as np
import jax
import jax.numpy as jnp
from jax import lax
from jax.experimental import pallas as pl
from jax.experimental.pallas import tpu as pltpu
from jax.experimental.pallas import tpu_sc as plsc

_NUM_FIELDS = 26
_EMBED_DIM = 16
_EMBED_OUT = _NUM_FIELDS * _EMBED_DIM
_B = 16384
_VOCAB = 100000 * _NUM_FIELDS
_N_E = _EMBED_OUT * _B
_N_L = _NUM_FIELDS * _B
_OFFS = np.arange(_NUM_FIELDS, dtype=np.int32) * 100000

_NC = 2
_NS = 16
_NW = _NC * _NS
_EPW = _N_E // _NW
_LPW = _N_L // _NW
_CHUNK = 4096
_LCHUNK = 3328

_S_MAT = np.tile(np.eye(_EMBED_DIM, dtype=np.float32), (_NUM_FIELDS, 1))


def _sc_compiler_params():
    cp = pltpu.CompilerParams(use_tc_tiling_on_sc=False)
    if "needs_layout_passes" in pltpu.CompilerParams.__dataclass_fields__:
        cp = dataclasses.replace(cp, needs_layout_passes=False)
    return cp


@functools.lru_cache(maxsize=1)
def _build_sc_gather_e():
    @functools.partial(
        pl.kernel,
        out_type=jax.ShapeDtypeStruct((_N_E,), jnp.float32),
        mesh=plsc.VectorSubcoreMesh(core_axis_name="c", subcore_axis_name="s"),
        scratch_types=[
            pltpu.VMEM((_CHUNK,), jnp.int32),
            pltpu.VMEM((_CHUNK,), jnp.int32),
            pltpu.VMEM((_CHUNK,), jnp.float32),
            pltpu.VMEM((_CHUNK,), jnp.float32),
            pltpu.SemaphoreType.DMA,
            pltpu.SemaphoreType.DMA,
        ],
        compiler_params=_sc_compiler_params(),
    )
    def _sc_gather_e(eidx_hbm, et_flat, gt_out,
                     eidx_v0, eidx_v1, eval_v0, eval_v1, sem0, sem1):
        wid = lax.axis_index("s") * _NC + lax.axis_index("c")
        ebase = wid * _EPW

        @pl.loop(0, _EPW, step=2 * _CHUNK)
        def _(off):
            s0 = ebase + off
            s1 = s0 + _CHUNK
            pltpu.sync_copy(eidx_hbm.at[pl.ds(s0, _CHUNK)], eidx_v0)
            cp0 = pltpu.async_copy(et_flat.at[eidx_v0], eval_v0, sem0)
            pltpu.sync_copy(eidx_hbm.at[pl.ds(s1, _CHUNK)], eidx_v1)
            cp1 = pltpu.async_copy(et_flat.at[eidx_v1], eval_v1, sem1)
            cp0.wait()
            pltpu.sync_copy(eval_v0, gt_out.at[pl.ds(s0, _CHUNK)])
            cp1.wait()
            pltpu.sync_copy(eval_v1, gt_out.at[pl.ds(s1, _CHUNK)])

    return _sc_gather_e


@functools.lru_cache(maxsize=1)
def _build_sc_gather_l():
    @functools.partial(
        pl.kernel,
        out_type=jax.ShapeDtypeStruct((_N_L,), jnp.float32),
        mesh=plsc.VectorSubcoreMesh(core_axis_name="c", subcore_axis_name="s"),
        scratch_types=[
            pltpu.VMEM((_LCHUNK,), jnp.int32),
            pltpu.VMEM((_LCHUNK,), jnp.float32),
            pltpu.SemaphoreType.DMA,
        ],
        compiler_params=_sc_compiler_params(),
    )
    def _sc_gather_l(lidx_hbm, l_flat, lval_out, lidx_v, lval_v, sem):
        wid = lax.axis_index("s") * _NC + lax.axis_index("c")
        lbase = wid * _LPW

        @pl.loop(0, _LPW, step=_LCHUNK)
        def _(off):
            start = lbase + off
            pltpu.sync_copy(lidx_hbm.at[pl.ds(start, _LCHUNK)], lidx_v)
            pltpu.async_copy(l_flat.at[lidx_v], lval_v, sem).wait()
            pltpu.sync_copy(lval_v, lval_out.at[pl.ds(start, _LCHUNK)])

    return _sc_gather_l


_VMAIN = _VOCAB - 64
_TBASE = _EMBED_DIM * _VMAIN


_CBLK = 262144
_CGRID = 10
_CLAST = _VMAIN - (_CGRID - 1) * _CBLK


def _compact_body(et_ref, tail_ref, out_ref, sem):
    i = pl.program_id(0)

    def emit(size):
        copies = [
            pltpu.make_async_copy(
                et_ref.at[d, pl.ds(0, size)],
                out_ref.at[pl.ds(d * _VMAIN + i * _CBLK, size)],
                sem,
            )
            for d in range(_EMBED_DIM)
        ]
        for c in copies:
            c.start()
        for c in copies:
            c.wait()

    @pl.when(i < _CGRID - 1)
    def _():
        emit(_CBLK)

    @pl.when(i == _CGRID - 1)
    def _():
        emit(_CLAST)

    @pl.when(i == 0)
    def _():
        c = pltpu.make_async_copy(tail_ref, out_ref.at[pl.ds(_TBASE, 1024)], sem)
        c.start()
        c.wait()


def _compact_et(ET, tail_flat):
    return pl.pallas_call(
        _compact_body,
        grid=(_CGRID,),
        in_specs=[pl.BlockSpec((_EMBED_DIM, _CBLK), lambda i: (0, i)),
                  pl.BlockSpec(memory_space=pl.ANY)],
        out_specs=pl.BlockSpec(memory_space=pl.ANY),
        out_shape=jax.ShapeDtypeStruct((_EMBED_DIM * _VOCAB,), jnp.float32),
        scratch_shapes=[pltpu.SemaphoreType.DMA],
    )(ET, tail_flat)


_BLK = 2048
_NB = _B // _BLK
_CONTRACT0 = (((0,), (0,)), ((), ()))


def _tc_body(gt_ref, lval_ref, W1_ref, b1_ref, g1_ref, be1_ref,
             W2_ref, b2_ref, g2_ref, be2_ref, W3_ref, sc_ref, S_ref,
             out_ref, h1_s, base_s):
    i = pl.program_id(0)
    M = gt_ref[...]
    h1 = lax.dot_general(M, W1_ref[...], _CONTRACT0,
                         preferred_element_type=jnp.float32) + b1_ref[...]
    h1_s[pl.ds(i * _BLK, _BLK), :] = h1

    s = lax.dot_general(M, S_ref[...], _CONTRACT0,
                        preferred_element_type=jnp.float32)
    fm = 0.5 * (jnp.sum(s * s, axis=1) - jnp.sum(M * M, axis=0))
    lin = jnp.sum(lval_ref[...], axis=0)
    base_s[pl.ds(i * _BLK, _BLK)] = lin + fm + sc_ref[0]

    @pl.when(i == _NB - 1)
    def _():
        H1 = h1_s[...]
        mu1 = jnp.mean(H1, axis=0, keepdims=True)
        d1 = H1 - mu1
        var1 = jnp.mean(d1 * d1, axis=0, keepdims=True)
        a1 = g1_ref[...] * lax.rsqrt(var1 + 1e-5)
        N1 = jnp.maximum(d1 * a1 + be1_ref[...], 0.0)
        H2 = jnp.dot(N1, W2_ref[...], preferred_element_type=jnp.float32) + b2_ref[...]
        mu2 = jnp.mean(H2, axis=0, keepdims=True)
        d2 = H2 - mu2
        var2 = jnp.mean(d2 * d2, axis=0, keepdims=True)
        a2 = g2_ref[...] * lax.rsqrt(var2 + 1e-5)
        N2 = jnp.maximum(d2 * a2 + be2_ref[...], 0.0)
        mlp = jnp.dot(N2, W3_ref[...], preferred_element_type=jnp.float32)[:, 0]
        z = base_s[...] + mlp
        e = jnp.exp(-jnp.abs(z))
        out_ref[...] = jnp.where(z >= 0, 1.0 / (1.0 + e), e / (1.0 + e))


def _tc_mlp(GT, lvalT, W1, b1, g1, be1, W2, b2, g2, be2, W3, sc):
    full = lambda shape: pl.BlockSpec(shape, lambda i: tuple(0 for _ in shape))
    return pl.pallas_call(
        _tc_body,
        grid=(_NB,),
        in_specs=[
            pl.BlockSpec((_EMBED_OUT, _BLK), lambda i: (0, i)),
            pl.BlockSpec((_NUM_FIELDS, _BLK), lambda i: (0, i)),
            full((_EMBED_OUT, 128)),
            full((1, 128)),
            full((1, 128)),
            full((1, 128)),
            full((128, 128)),
            full((1, 128)),
            full((1, 128)),
            full((1, 128)),
            full((128, 1)),
            pl.BlockSpec(memory_space=pltpu.SMEM),
            full((_EMBED_OUT, _EMBED_DIM)),
        ],
        out_specs=pl.BlockSpec((_B,), lambda i: (0,)),
        out_shape=jax.ShapeDtypeStruct((_B,), jnp.float32),
        scratch_shapes=[
            pltpu.VMEM((_B, 128), jnp.float32),
            pltpu.VMEM((_B,), jnp.float32),
        ],
    )(GT, lvalT, W1, b1.reshape(1, 128), g1.reshape(1, 128), be1.reshape(1, 128),
      W2, b2.reshape(1, 128), g2.reshape(1, 128), be2.reshape(1, 128),
      W3, sc, jnp.asarray(_S_MAT))


def kernel(x, E, L, bias, W1, b1, g1, be1, W2, b2, g2, be2, W3, b3):
    idxT = x.T.astype(jnp.int32) + jnp.asarray(_OFFS)[:, None]
    d_off = jnp.arange(_EMBED_DIM, dtype=jnp.int32)[None, :, None]
    v = idxT[:, None, :]
    eidx = jnp.where(v < _VMAIN,
                     d_off * _VMAIN + v,
                     _TBASE + d_off * 64 + (v - _VMAIN))
    eidx = eidx.reshape(_N_E)
    lidx = idxT.reshape(_N_L)
    tail_flat = lax.slice(E, (_VMAIN, 0), (_VOCAB, _EMBED_DIM)).T.reshape(1024)
    et_flat = _compact_et(E.T, tail_flat)
    l_flat = L.reshape(_VOCAB)
    gt_flat = _build_sc_gather_e()(eidx, et_flat)
    lval_flat = _build_sc_gather_l()(lidx, l_flat)
    GT = gt_flat.reshape(_EMBED_OUT, _B)
    lvalT = lval_flat.reshape(_NUM_FIELDS, _B)
    sc = (bias + b3).reshape(1)
    return _tc_mlp(GT, lvalT, W1, b1, g1, be1, W2, b2, g2, be2, W3, sc)


# speedup vs baseline: 10.2765x; 1.1443x over previous
import dataclasses
import functools

import numpy as np
import jax
import jax.numpy as jnp
from jax import lax
from jax.experimental import pallas as pl
from jax.experimental.pallas import tpu as pltpu
from jax.experimental.pallas import tpu_sc as plsc

_NUM_FIELDS = 26
_EMBED_DIM = 16
_EMBED_OUT = _NUM_FIELDS * _EMBED_DIM
_B = 16384
_VOCAB = 100000 * _NUM_FIELDS
_N_E = _EMBED_OUT * _B
_N_L = _NUM_FIELDS * _B
_OFFS = np.arange(_NUM_FIELDS, dtype=np.int32) * 100000

_NC = 2
_NS = 16
_NW = _NC * _NS
_EPW = _N_E // _NW
_LPW = _N_L // _NW
_CHUNK = 8192
_LCHUNK = 3328

_S_MAT = np.tile(np.eye(_EMBED_DIM, dtype=np.float32), (_NUM_FIELDS, 1))


def _sc_compiler_params():
    cp = pltpu.CompilerParams(use_tc_tiling_on_sc=False)
    if "needs_layout_passes" in pltpu.CompilerParams.__dataclass_fields__:
        cp = dataclasses.replace(cp, needs_layout_passes=False)
    return cp


@functools.lru_cache(maxsize=1)
def _build_sc_gather_e():
    @functools.partial(
        pl.kernel,
        out_type=jax.ShapeDtypeStruct((_N_E,), jnp.float32),
        mesh=plsc.VectorSubcoreMesh(core_axis_name="c", subcore_axis_name="s"),
        scratch_types=[
            pltpu.VMEM((_CHUNK,), jnp.int32),
            pltpu.VMEM((_CHUNK,), jnp.int32),
            pltpu.VMEM((_CHUNK,), jnp.float32),
            pltpu.VMEM((_CHUNK,), jnp.float32),
            pltpu.SemaphoreType.DMA,
            pltpu.SemaphoreType.DMA,
        ],
        compiler_params=_sc_compiler_params(),
    )
    def _sc_gather_e(eidx_hbm, et_flat, gt_out,
                     eidx_v0, eidx_v1, eval_v0, eval_v1, sem0, sem1):
        wid = lax.axis_index("s") * _NC + lax.axis_index("c")
        ebase = wid * _EPW

        @pl.loop(0, _EPW, step=2 * _CHUNK)
        def _(off):
            s0 = ebase + off
            s1 = s0 + _CHUNK
            pltpu.sync_copy(eidx_hbm.at[pl.ds(s0, _CHUNK)], eidx_v0)
            cp0 = pltpu.async_copy(et_flat.at[eidx_v0], eval_v0, sem0)
            pltpu.sync_copy(eidx_hbm.at[pl.ds(s1, _CHUNK)], eidx_v1)
            cp1 = pltpu.async_copy(et_flat.at[eidx_v1], eval_v1, sem1)
            cp0.wait()
            pltpu.sync_copy(eval_v0, gt_out.at[pl.ds(s0, _CHUNK)])
            cp1.wait()
            pltpu.sync_copy(eval_v1, gt_out.at[pl.ds(s1, _CHUNK)])

    return _sc_gather_e


@functools.lru_cache(maxsize=1)
def _build_sc_gather_l():
    @functools.partial(
        pl.kernel,
        out_type=jax.ShapeDtypeStruct((_N_L,), jnp.float32),
        mesh=plsc.VectorSubcoreMesh(core_axis_name="c", subcore_axis_name="s"),
        scratch_types=[
            pltpu.VMEM((_LCHUNK,), jnp.int32),
            pltpu.VMEM((_LCHUNK,), jnp.float32),
            pltpu.SemaphoreType.DMA,
        ],
        compiler_params=_sc_compiler_params(),
    )
    def _sc_gather_l(lidx_hbm, l_flat, lval_out, lidx_v, lval_v, sem):
        wid = lax.axis_index("s") * _NC + lax.axis_index("c")
        lbase = wid * _LPW

        @pl.loop(0, _LPW, step=_LCHUNK)
        def _(off):
            start = lbase + off
            pltpu.sync_copy(lidx_hbm.at[pl.ds(start, _LCHUNK)], lidx_v)
            pltpu.async_copy(l_flat.at[lidx_v], lval_v, sem).wait()
            pltpu.sync_copy(lval_v, lval_out.at[pl.ds(start, _LCHUNK)])

    return _sc_gather_l


_VMAIN = _VOCAB - 64
_TBASE = _EMBED_DIM * _VMAIN


_CBLK = 262144
_CGRID = 10
_CLAST = _VMAIN - (_CGRID - 1) * _CBLK


def _compact_body(et_ref, lt_ref, tail_ref, tail_l_ref, out_ref, lout_ref, sem):
    i = pl.program_id(0)

    def emit(size):
        copies = [
            pltpu.make_async_copy(
                et_ref.at[d, pl.ds(0, size)],
                out_ref.at[pl.ds(d * _VMAIN + i * _CBLK, size)],
                sem,
            )
            for d in range(_EMBED_DIM)
        ]
        copies.append(
            pltpu.make_async_copy(
                lt_ref.at[0, pl.ds(0, size)],
                lout_ref.at[pl.ds(i * _CBLK, size)],
                sem,
            )
        )
        for c in copies:
            c.start()
        for c in copies:
            c.wait()

    @pl.when(i < _CGRID - 1)
    def _():
        emit(_CBLK)

    @pl.when(i == _CGRID - 1)
    def _():
        emit(_CLAST)

    @pl.when(i == 0)
    def _():
        c1 = pltpu.make_async_copy(tail_ref, out_ref.at[pl.ds(_TBASE, 1024)], sem)
        c2 = pltpu.make_async_copy(tail_l_ref, lout_ref.at[pl.ds(_VMAIN, 128)], sem)
        c1.start()
        c2.start()
        c1.wait()
        c2.wait()


def _compact_tables(ET, LT, tail_flat, tail_l):
    return pl.pallas_call(
        _compact_body,
        grid=(_CGRID,),
        in_specs=[pl.BlockSpec((_EMBED_DIM, _CBLK), lambda i: (0, i)),
                  pl.BlockSpec((1, _CBLK), lambda i: (0, i)),
                  pl.BlockSpec(memory_space=pl.ANY),
                  pl.BlockSpec(memory_space=pl.ANY)],
        out_specs=[pl.BlockSpec(memory_space=pl.ANY),
                   pl.BlockSpec(memory_space=pl.ANY)],
        out_shape=[jax.ShapeDtypeStruct((_EMBED_DIM * _VOCAB,), jnp.float32),
                   jax.ShapeDtypeStruct((_VMAIN + 128,), jnp.float32)],
        scratch_shapes=[pltpu.SemaphoreType.DMA],
    )(ET, LT, tail_flat, tail_l)


_BLK = 2048
_NB = _B // _BLK
_CONTRACT0 = (((0,), (0,)), ((), ()))


def _tc_body(gt_ref, lval_ref, W1_ref, b1_ref, g1_ref, be1_ref,
             W2_ref, b2_ref, g2_ref, be2_ref, W3_ref, sc_ref, S_ref,
             out_ref, h1_s, base_s):
    i = pl.program_id(0)
    M = gt_ref[...]
    h1 = lax.dot_general(M, W1_ref[...], _CONTRACT0,
                         preferred_element_type=jnp.float32) + b1_ref[...]
    h1_s[pl.ds(i * _BLK, _BLK), :] = h1

    s = lax.dot_general(M, S_ref[...], _CONTRACT0,
                        preferred_element_type=jnp.float32)
    fm = 0.5 * (jnp.sum(s * s, axis=1) - jnp.sum(M * M, axis=0))
    lin = jnp.sum(lval_ref[...], axis=0)
    base_s[pl.ds(i * _BLK, _BLK)] = lin + fm + sc_ref[0]

    @pl.when(i == _NB - 1)
    def _():
        H1 = h1_s[...]
        mu1 = jnp.mean(H1, axis=0, keepdims=True)
        d1 = H1 - mu1
        var1 = jnp.mean(d1 * d1, axis=0, keepdims=True)
        a1 = g1_ref[...] * lax.rsqrt(var1 + 1e-5)
        N1 = jnp.maximum(d1 * a1 + be1_ref[...], 0.0)
        H2 = jnp.dot(N1, W2_ref[...], preferred_element_type=jnp.float32) + b2_ref[...]
        mu2 = jnp.mean(H2, axis=0, keepdims=True)
        d2 = H2 - mu2
        var2 = jnp.mean(d2 * d2, axis=0, keepdims=True)
        a2 = g2_ref[...] * lax.rsqrt(var2 + 1e-5)
        N2 = jnp.maximum(d2 * a2 + be2_ref[...], 0.0)
        mlp = jnp.dot(N2, W3_ref[...], preferred_element_type=jnp.float32)[:, 0]
        z = base_s[...] + mlp
        e = jnp.exp(-jnp.abs(z))
        out_ref[...] = jnp.where(z >= 0, 1.0 / (1.0 + e), e / (1.0 + e))


def _tc_mlp(GT, lvalT, W1, b1, g1, be1, W2, b2, g2, be2, W3, sc):
    full = lambda shape: pl.BlockSpec(shape, lambda i: tuple(0 for _ in shape))
    return pl.pallas_call(
        _tc_body,
        grid=(_NB,),
        in_specs=[
            pl.BlockSpec((_EMBED_OUT, _BLK), lambda i: (0, i)),
            pl.BlockSpec((_NUM_FIELDS, _BLK), lambda i: (0, i)),
            full((_EMBED_OUT, 128)),
            full((1, 128)),
            full((1, 128)),
            full((1, 128)),
            full((128, 128)),
            full((1, 128)),
            full((1, 128)),
            full((1, 128)),
            full((128, 1)),
            pl.BlockSpec(memory_space=pltpu.SMEM),
            full((_EMBED_OUT, _EMBED_DIM)),
        ],
        out_specs=pl.BlockSpec((_B,), lambda i: (0,)),
        out_shape=jax.ShapeDtypeStruct((_B,), jnp.float32),
        scratch_shapes=[
            pltpu.VMEM((_B, 128), jnp.float32),
            pltpu.VMEM((_B,), jnp.float32),
        ],
    )(GT, lvalT, W1, b1.reshape(1, 128), g1.reshape(1, 128), be1.reshape(1, 128),
      W2, b2.reshape(1, 128), g2.reshape(1, 128), be2.reshape(1, 128),
      W3, sc, jnp.asarray(_S_MAT))


def kernel(x, E, L, bias, W1, b1, g1, be1, W2, b2, g2, be2, W3, b3):
    idxT = x.T.astype(jnp.int32) + jnp.asarray(_OFFS)[:, None]
    d_off = jnp.arange(_EMBED_DIM, dtype=jnp.int32)[None, :, None]
    v = idxT[:, None, :]
    eidx = jnp.where(v < _VMAIN,
                     d_off * _VMAIN + v,
                     _TBASE + d_off * 64 + (v - _VMAIN))
    eidx = eidx.reshape(_N_E)
    lidx = idxT.reshape(_N_L)
    tail_flat = lax.slice(E, (_VMAIN, 0), (_VOCAB, _EMBED_DIM)).T.reshape(1024)
    tail_l = jnp.concatenate(
        [lax.slice(L, (_VMAIN, 0), (_VOCAB, 1))[:, 0],
         jnp.zeros((64,), jnp.float32)])
    et_flat, l_flat = _compact_tables(E.T, L.T, tail_flat, tail_l)
    gt_flat = _build_sc_gather_e()(eidx, et_flat)
    lval_flat = _build_sc_gather_l()(lidx, l_flat)
    GT = gt_flat.reshape(_EMBED_OUT, _B)
    lvalT = lval_flat.reshape(_NUM_FIELDS, _B)
    sc = (bias + b3).reshape(1)
    return _tc_mlp(GT, lvalT, W1, b1, g1, be1, W2, b2, g2, be2, W3, sc)


# speedup vs baseline: 10.8090x; 1.0518x over previous
import dataclasses
import functools

import numpy as np
import jax
import jax.numpy as jnp
from jax import lax
from jax.experimental import pallas as pl
from jax.experimental.pallas import tpu as pltpu
from jax.experimental.pallas import tpu_sc as plsc

_NUM_FIELDS = 26
_EMBED_DIM = 16
_EMBED_OUT = _NUM_FIELDS * _EMBED_DIM
_B = 16384
_VOCAB = 100000 * _NUM_FIELDS
_N_E = _EMBED_OUT * _B
_N_L = _NUM_FIELDS * _B
_OFFS = np.arange(_NUM_FIELDS, dtype=np.int32) * 100000

_NC = 2
_NS = 16
_NW = _NC * _NS
_EPW = _N_E // _NW
_LPW = _N_L // _NW
_CHUNK = 4096
_NFLIGHT = 4
_LCHUNK = 3328

_S_MAT = np.tile(np.eye(_EMBED_DIM, dtype=np.float32), (_NUM_FIELDS, 1))


def _sc_compiler_params():
    cp = pltpu.CompilerParams(use_tc_tiling_on_sc=False)
    if "needs_layout_passes" in pltpu.CompilerParams.__dataclass_fields__:
        cp = dataclasses.replace(cp, needs_layout_passes=False)
    return cp


@functools.lru_cache(maxsize=1)
def _build_sc_gather_e():
    @functools.partial(
        pl.kernel,
        out_type=jax.ShapeDtypeStruct((_N_E,), jnp.float32),
        mesh=plsc.VectorSubcoreMesh(core_axis_name="c", subcore_axis_name="s"),
        scratch_types=(
            [pltpu.VMEM((_CHUNK,), jnp.int32) for _ in range(_NFLIGHT)]
            + [pltpu.VMEM((_CHUNK,), jnp.float32) for _ in range(_NFLIGHT)]
            + [pltpu.SemaphoreType.DMA for _ in range(_NFLIGHT)]
        ),
        compiler_params=_sc_compiler_params(),
    )
    def _sc_gather_e(eidx_hbm, et_flat, gt_out, *scratch):
        idx_v = scratch[:_NFLIGHT]
        val_v = scratch[_NFLIGHT:2 * _NFLIGHT]
        sems = scratch[2 * _NFLIGHT:]
        wid = lax.axis_index("s") * _NC + lax.axis_index("c")
        ebase = wid * _EPW

        @pl.loop(0, _EPW, step=_NFLIGHT * _CHUNK)
        def _(off):
            cps = []
            for n in range(_NFLIGHT):
                s = ebase + off + n * _CHUNK
                pltpu.sync_copy(eidx_hbm.at[pl.ds(s, _CHUNK)], idx_v[n])
                cps.append(pltpu.async_copy(et_flat.at[idx_v[n]], val_v[n], sems[n]))
            for n in range(_NFLIGHT):
                cps[n].wait()
                pltpu.sync_copy(val_v[n], gt_out.at[pl.ds(ebase + off + n * _CHUNK, _CHUNK)])

    return _sc_gather_e


@functools.lru_cache(maxsize=1)
def _build_sc_gather_l():
    @functools.partial(
        pl.kernel,
        out_type=jax.ShapeDtypeStruct((_N_L,), jnp.float32),
        mesh=plsc.VectorSubcoreMesh(core_axis_name="c", subcore_axis_name="s"),
        scratch_types=[
            pltpu.VMEM((_LCHUNK,), jnp.int32),
            pltpu.VMEM((_LCHUNK,), jnp.float32),
            pltpu.SemaphoreType.DMA,
        ],
        compiler_params=_sc_compiler_params(),
    )
    def _sc_gather_l(lidx_hbm, l_flat, lval_out, lidx_v, lval_v, sem):
        wid = lax.axis_index("s") * _NC + lax.axis_index("c")
        lbase = wid * _LPW

        @pl.loop(0, _LPW, step=_LCHUNK)
        def _(off):
            start = lbase + off
            pltpu.sync_copy(lidx_hbm.at[pl.ds(start, _LCHUNK)], lidx_v)
            pltpu.async_copy(l_flat.at[lidx_v], lval_v, sem).wait()
            pltpu.sync_copy(lval_v, lval_out.at[pl.ds(start, _LCHUNK)])

    return _sc_gather_l


_VMAIN = _VOCAB - 64
_TBASE = _EMBED_DIM * _VMAIN


_CBLK = 262144
_CGRID = 10
_CLAST = _VMAIN - (_CGRID - 1) * _CBLK


def _compact_body(et_ref, lt_ref, tail_ref, tail_l_ref, out_ref, lout_ref, sem):
    i = pl.program_id(0)

    def emit(size):
        copies = [
            pltpu.make_async_copy(
                et_ref.at[d, pl.ds(0, size)],
                out_ref.at[pl.ds(d * _VMAIN + i * _CBLK, size)],
                sem,
            )
            for d in range(_EMBED_DIM)
        ]
        copies.append(
            pltpu.make_async_copy(
                lt_ref.at[0, pl.ds(0, size)],
                lout_ref.at[pl.ds(i * _CBLK, size)],
                sem,
            )
        )
        for c in copies:
            c.start()
        for c in copies:
            c.wait()

    @pl.when(i < _CGRID - 1)
    def _():
        emit(_CBLK)

    @pl.when(i == _CGRID - 1)
    def _():
        emit(_CLAST)

    @pl.when(i == 0)
    def _():
        c1 = pltpu.make_async_copy(tail_ref, out_ref.at[pl.ds(_TBASE, 1024)], sem)
        c2 = pltpu.make_async_copy(tail_l_ref, lout_ref.at[pl.ds(_VMAIN, 128)], sem)
        c1.start()
        c2.start()
        c1.wait()
        c2.wait()


def _compact_tables(ET, LT, tail_flat, tail_l):
    return pl.pallas_call(
        _compact_body,
        grid=(_CGRID,),
        in_specs=[pl.BlockSpec((_EMBED_DIM, _CBLK), lambda i: (0, i)),
                  pl.BlockSpec((1, _CBLK), lambda i: (0, i)),
                  pl.BlockSpec(memory_space=pl.ANY),
                  pl.BlockSpec(memory_space=pl.ANY)],
        out_specs=[pl.BlockSpec(memory_space=pl.ANY),
                   pl.BlockSpec(memory_space=pl.ANY)],
        out_shape=[jax.ShapeDtypeStruct((_EMBED_DIM * _VOCAB,), jnp.float32),
                   jax.ShapeDtypeStruct((_VMAIN + 128,), jnp.float32)],
        scratch_shapes=[pltpu.SemaphoreType.DMA],
    )(ET, LT, tail_flat, tail_l)


_BLK = 2048
_NB = _B // _BLK
_CONTRACT0 = (((0,), (0,)), ((), ()))


def _tc_body(gt_ref, lval_ref, W1_ref, b1_ref, g1_ref, be1_ref,
             W2_ref, b2_ref, g2_ref, be2_ref, W3_ref, sc_ref, S_ref,
             out_ref, h1_s, base_s):
    i = pl.program_id(0)
    M = gt_ref[...].reshape(_EMBED_OUT, _BLK)
    h1 = lax.dot_general(M, W1_ref[...], _CONTRACT0,
                         preferred_element_type=jnp.float32) + b1_ref[...]
    h1_s[pl.ds(i * _BLK, _BLK), :] = h1

    s = lax.dot_general(M, S_ref[...], _CONTRACT0,
                        preferred_element_type=jnp.float32)
    fm = 0.5 * (jnp.sum(s * s, axis=1) - jnp.sum(M * M, axis=0))
    lin = jnp.sum(lval_ref[...], axis=0)
    base_s[pl.ds(i * _BLK, _BLK)] = lin + fm + sc_ref[0]

    @pl.when(i == _NB - 1)
    def _():
        H1 = h1_s[...]
        mu1 = jnp.mean(H1, axis=0, keepdims=True)
        d1 = H1 - mu1
        var1 = jnp.mean(d1 * d1, axis=0, keepdims=True)
        a1 = g1_ref[...] * lax.rsqrt(var1 + 1e-5)
        N1 = jnp.maximum(d1 * a1 + be1_ref[...], 0.0)
        H2 = jnp.dot(N1, W2_ref[...], preferred_element_type=jnp.float32) + b2_ref[...]
        mu2 = jnp.mean(H2, axis=0, keepdims=True)
        d2 = H2 - mu2
        var2 = jnp.mean(d2 * d2, axis=0, keepdims=True)
        a2 = g2_ref[...] * lax.rsqrt(var2 + 1e-5)
        N2 = jnp.maximum(d2 * a2 + be2_ref[...], 0.0)
        mlp = jnp.dot(N2, W3_ref[...], preferred_element_type=jnp.float32)[:, 0]
        z = base_s[...] + mlp
        e = jnp.exp(-jnp.abs(z))
        out_ref[...] = jnp.where(z >= 0, 1.0 / (1.0 + e), e / (1.0 + e))


def _tc_mlp(GT, lvalT, W1, b1, g1, be1, W2, b2, g2, be2, W3, sc):
    full = lambda shape: pl.BlockSpec(shape, lambda i: tuple(0 for _ in shape))
    return pl.pallas_call(
        _tc_body,
        grid=(_NB,),
        in_specs=[
            pl.BlockSpec((_EMBED_OUT, _BLK // 128, 128), lambda i: (0, i, 0)),
            pl.BlockSpec((_NUM_FIELDS, _BLK), lambda i: (0, i)),
            full((_EMBED_OUT, 128)),
            full((1, 128)),
            full((1, 128)),
            full((1, 128)),
            full((128, 128)),
            full((1, 128)),
            full((1, 128)),
            full((1, 128)),
            full((128, 1)),
            pl.BlockSpec(memory_space=pltpu.SMEM),
            full((_EMBED_OUT, _EMBED_DIM)),
        ],
        out_specs=pl.BlockSpec((_B,), lambda i: (0,)),
        out_shape=jax.ShapeDtypeStruct((_B,), jnp.float32),
        scratch_shapes=[
            pltpu.VMEM((_B, 128), jnp.float32),
            pltpu.VMEM((_B,), jnp.float32),
        ],
    )(GT, lvalT, W1, b1.reshape(1, 128), g1.reshape(1, 128), be1.reshape(1, 128),
      W2, b2.reshape(1, 128), g2.reshape(1, 128), be2.reshape(1, 128),
      W3, sc, jnp.asarray(_S_MAT))


def kernel(x, E, L, bias, W1, b1, g1, be1, W2, b2, g2, be2, W3, b3):
    idxT = x.T.astype(jnp.int32) + jnp.asarray(_OFFS)[:, None]
    d_off = jnp.arange(_EMBED_DIM, dtype=jnp.int32)[None, :, None]
    v = idxT[:, None, :]
    eidx = jnp.where(v < _VMAIN,
                     d_off * _VMAIN + v,
                     _TBASE + d_off * 64 + (v - _VMAIN))
    eidx = eidx.reshape(_N_E)
    lidx = idxT.reshape(_N_L)
    tail_flat = lax.slice(E, (_VMAIN, 0), (_VOCAB, _EMBED_DIM)).T.reshape(1024)
    tail_l = jnp.concatenate(
        [lax.slice(L, (_VMAIN, 0), (_VOCAB, 1))[:, 0],
         jnp.zeros((64,), jnp.float32)])
    et_flat, l_flat = _compact_tables(E.T, L.T, tail_flat, tail_l)
    gt_flat = _build_sc_gather_e()(eidx, et_flat)
    lval_flat = _build_sc_gather_l()(lidx, l_flat)
    GT = gt_flat.reshape(_EMBED_OUT, _B // 128, 128)
    lvalT = lval_flat.reshape(_NUM_FIELDS, _B)
    sc = (bias + b3).reshape(1)
    return _tc_mlp(GT, lvalT, W1, b1, g1, be1, W2, b2, g2, be2, W3, sc)


# speedup vs baseline: 11.0526x; 1.0225x over previous
import dataclasses
import functools

import numpy as np
import jax
import jax.numpy as jnp
from jax import lax
from jax.experimental import pallas as pl
from jax.experimental.pallas import tpu as pltpu
from jax.experimental.pallas import tpu_sc as plsc

_NUM_FIELDS = 26
_EMBED_DIM = 16
_EMBED_OUT = _NUM_FIELDS * _EMBED_DIM
_B = 16384
_VOCAB = 100000 * _NUM_FIELDS
_N_E = _EMBED_OUT * _B
_N_L = _NUM_FIELDS * _B
_OFFS = np.arange(_NUM_FIELDS, dtype=np.int32) * 100000

_NC = 2
_NS = 16
_NW = _NC * _NS
_EPW = _N_E // _NW
_LPW = _N_L // _NW
_CHUNK = 4096
_NFLIGHT = 2
_LCHUNK = 3328

_S_MAT = np.tile(np.eye(_EMBED_DIM, dtype=np.float32), (_NUM_FIELDS, 1))
_PERM = np.array([16 * (jp % _NUM_FIELDS) + jp // _NUM_FIELDS
                  for jp in range(_EMBED_OUT)], dtype=np.int32)
_S_MAT_P = _S_MAT[_PERM]
_HALF = _EMBED_OUT // 2
_N_EH = _N_E // 2
_EPWH = _N_EH // _NW


def _sc_compiler_params():
    cp = pltpu.CompilerParams(use_tc_tiling_on_sc=False)
    if "needs_layout_passes" in pltpu.CompilerParams.__dataclass_fields__:
        cp = dataclasses.replace(cp, needs_layout_passes=False)
    return cp


@functools.lru_cache(maxsize=2)
def _build_sc_gather_e(n_out):
    per_w = n_out // _NW

    @functools.partial(
        pl.kernel,
        out_type=jax.ShapeDtypeStruct((n_out,), jnp.float32),
        mesh=plsc.VectorSubcoreMesh(core_axis_name="c", subcore_axis_name="s"),
        scratch_types=(
            [pltpu.VMEM((_CHUNK,), jnp.int32) for _ in range(_NFLIGHT)]
            + [pltpu.VMEM((_CHUNK,), jnp.float32) for _ in range(_NFLIGHT)]
            + [pltpu.SemaphoreType.DMA for _ in range(_NFLIGHT)]
        ),
        compiler_params=_sc_compiler_params(),
    )
    def _sc_gather_e(eidx_hbm, et_flat, gt_out, *scratch):
        idx_v = scratch[:_NFLIGHT]
        val_v = scratch[_NFLIGHT:2 * _NFLIGHT]
        sems = scratch[2 * _NFLIGHT:]
        wid = lax.axis_index("s") * _NC + lax.axis_index("c")
        ebase = wid * per_w

        @pl.loop(0, per_w, step=_NFLIGHT * _CHUNK)
        def _(off):
            cps = []
            for n in range(_NFLIGHT):
                s = ebase + off + n * _CHUNK
                pltpu.sync_copy(eidx_hbm.at[pl.ds(s, _CHUNK)], idx_v[n])
                cps.append(pltpu.async_copy(et_flat.at[idx_v[n]], val_v[n], sems[n]))
            for n in range(_NFLIGHT):
                cps[n].wait()
                pltpu.sync_copy(val_v[n], gt_out.at[pl.ds(ebase + off + n * _CHUNK, _CHUNK)])

    return _sc_gather_e


@functools.lru_cache(maxsize=1)
def _build_sc_gather_l():
    @functools.partial(
        pl.kernel,
        out_type=jax.ShapeDtypeStruct((_N_L,), jnp.float32),
        mesh=plsc.VectorSubcoreMesh(core_axis_name="c", subcore_axis_name="s"),
        scratch_types=[
            pltpu.VMEM((_LCHUNK,), jnp.int32),
            pltpu.VMEM((_LCHUNK,), jnp.float32),
            pltpu.SemaphoreType.DMA,
        ],
        compiler_params=_sc_compiler_params(),
    )
    def _sc_gather_l(lidx_hbm, l_flat, lval_out, lidx_v, lval_v, sem):
        wid = lax.axis_index("s") * _NC + lax.axis_index("c")
        lbase = wid * _LPW

        @pl.loop(0, _LPW, step=_LCHUNK)
        def _(off):
            start = lbase + off
            pltpu.sync_copy(lidx_hbm.at[pl.ds(start, _LCHUNK)], lidx_v)
            pltpu.async_copy(l_flat.at[lidx_v], lval_v, sem).wait()
            pltpu.sync_copy(lval_v, lval_out.at[pl.ds(start, _LCHUNK)])

    return _sc_gather_l


_VMAIN = _VOCAB - 64
_TBASE = _EMBED_DIM * _VMAIN


_CBLK = 262144
_CGRID = 10
_CLAST = _VMAIN - (_CGRID - 1) * _CBLK


_HTBASE = 8 * _VMAIN


def _compact_half_lo(ET, LT, tail_lo, tail_l):
    def body(et_ref, lt_ref, tail_ref, tail_l_ref, out_ref, lout_ref, sem):
        i = pl.program_id(0)

        def emit(size):
            copies = [
                pltpu.make_async_copy(
                    et_ref.at[d, pl.ds(0, size)],
                    out_ref.at[pl.ds(d * _VMAIN + i * _CBLK, size)],
                    sem,
                )
                for d in range(8)
            ]
            copies.append(
                pltpu.make_async_copy(
                    lt_ref.at[0, pl.ds(0, size)],
                    lout_ref.at[pl.ds(i * _CBLK, size)],
                    sem,
                )
            )
            for c in copies:
                c.start()
            for c in copies:
                c.wait()

        @pl.when(i < _CGRID - 1)
        def _():
            emit(_CBLK)

        @pl.when(i == _CGRID - 1)
        def _():
            emit(_CLAST)

        @pl.when(i == 0)
        def _():
            c1 = pltpu.make_async_copy(tail_ref, out_ref.at[pl.ds(_HTBASE, 512)], sem)
            c2 = pltpu.make_async_copy(tail_l_ref, lout_ref.at[pl.ds(_VMAIN, 128)], sem)
            c1.start()
            c2.start()
            c1.wait()
            c2.wait()

    return pl.pallas_call(
        body,
        grid=(_CGRID,),
        in_specs=[pl.BlockSpec((8, _CBLK), lambda i: (0, i)),
                  pl.BlockSpec((1, _CBLK), lambda i: (0, i)),
                  pl.BlockSpec(memory_space=pl.ANY),
                  pl.BlockSpec(memory_space=pl.ANY)],
        out_specs=[pl.BlockSpec(memory_space=pl.ANY),
                   pl.BlockSpec(memory_space=pl.ANY)],
        out_shape=[jax.ShapeDtypeStruct((_HTBASE + 512,), jnp.float32),
                   jax.ShapeDtypeStruct((_VMAIN + 128,), jnp.float32)],
        scratch_shapes=[pltpu.SemaphoreType.DMA],
    )(ET, LT, tail_lo, tail_l)


def _compact_half_hi(ET, tail_hi):
    def body(et_ref, tail_ref, out_ref, sem):
        i = pl.program_id(0)

        def emit(size):
            copies = [
                pltpu.make_async_copy(
                    et_ref.at[d, pl.ds(0, size)],
                    out_ref.at[pl.ds(d * _VMAIN + i * _CBLK, size)],
                    sem,
                )
                for d in range(8)
            ]
            for c in copies:
                c.start()
            for c in copies:
                c.wait()

        @pl.when(i < _CGRID - 1)
        def _():
            emit(_CBLK)

        @pl.when(i == _CGRID - 1)
        def _():
            emit(_CLAST)

        @pl.when(i == 0)
        def _():
            c1 = pltpu.make_async_copy(tail_ref, out_ref.at[pl.ds(_HTBASE, 512)], sem)
            c1.start()
            c1.wait()

    return pl.pallas_call(
        body,
        grid=(_CGRID,),
        in_specs=[pl.BlockSpec((8, _CBLK), lambda i: (1, i)),
                  pl.BlockSpec(memory_space=pl.ANY)],
        out_specs=pl.BlockSpec(memory_space=pl.ANY),
        out_shape=jax.ShapeDtypeStruct((_HTBASE + 512,), jnp.float32),
        scratch_shapes=[pltpu.SemaphoreType.DMA],
    )(ET, tail_hi)


_BLK = 2048
_NB = _B // _BLK
_CONTRACT0 = (((0,), (0,)), ((), ()))


def _tc_body(gt_lo_ref, gt_hi_ref, lval_ref, W1_ref, b1_ref, g1_ref, be1_ref,
             W2_ref, b2_ref, g2_ref, be2_ref, W3_ref, sc_ref, S_ref,
             out_ref, h1_s, base_s):
    i = pl.program_id(0)
    Ml = gt_lo_ref[...].reshape(_HALF, _BLK)
    Mh = gt_hi_ref[...].reshape(_HALF, _BLK)
    h1 = (lax.dot_general(Ml, W1_ref[0:_HALF, :], _CONTRACT0,
                          preferred_element_type=jnp.float32)
          + lax.dot_general(Mh, W1_ref[_HALF:, :], _CONTRACT0,
                            preferred_element_type=jnp.float32)
          + b1_ref[...])
    h1_s[pl.ds(i * _BLK, _BLK), :] = h1

    s = (lax.dot_general(Ml, S_ref[0:_HALF, :], _CONTRACT0,
                         preferred_element_type=jnp.float32)
         + lax.dot_general(Mh, S_ref[_HALF:, :], _CONTRACT0,
                           preferred_element_type=jnp.float32))
    fm = 0.5 * (jnp.sum(s * s, axis=1)
                - jnp.sum(Ml * Ml, axis=0) - jnp.sum(Mh * Mh, axis=0))
    lin = jnp.sum(lval_ref[...], axis=0)
    base_s[pl.ds(i * _BLK, _BLK)] = lin + fm + sc_ref[0]

    @pl.when(i == _NB - 1)
    def _():
        H1 = h1_s[...]
        mu1 = jnp.mean(H1, axis=0, keepdims=True)
        d1 = H1 - mu1
        var1 = jnp.mean(d1 * d1, axis=0, keepdims=True)
        a1 = g1_ref[...] * lax.rsqrt(var1 + 1e-5)
        N1 = jnp.maximum(d1 * a1 + be1_ref[...], 0.0)
        H2 = jnp.dot(N1, W2_ref[...], preferred_element_type=jnp.float32) + b2_ref[...]
        mu2 = jnp.mean(H2, axis=0, keepdims=True)
        d2 = H2 - mu2
        var2 = jnp.mean(d2 * d2, axis=0, keepdims=True)
        a2 = g2_ref[...] * lax.rsqrt(var2 + 1e-5)
        N2 = jnp.maximum(d2 * a2 + be2_ref[...], 0.0)
        mlp = jnp.dot(N2, W3_ref[...], preferred_element_type=jnp.float32)[:, 0]
        z = base_s[...] + mlp
        e = jnp.exp(-jnp.abs(z))
        out_ref[...] = jnp.where(z >= 0, 1.0 / (1.0 + e), e / (1.0 + e))


def _tc_mlp(GTlo, GThi, lvalT, W1, b1, g1, be1, W2, b2, g2, be2, W3, sc):
    full = lambda shape: pl.BlockSpec(shape, lambda i: tuple(0 for _ in shape))
    return pl.pallas_call(
        _tc_body,
        grid=(_NB,),
        in_specs=[
            pl.BlockSpec((_HALF, _BLK // 128, 128), lambda i: (0, i, 0)),
            pl.BlockSpec((_HALF, _BLK // 128, 128), lambda i: (0, i, 0)),
            pl.BlockSpec((_NUM_FIELDS, _BLK), lambda i: (0, i)),
            full((_EMBED_OUT, 128)),
            full((1, 128)),
            full((1, 128)),
            full((1, 128)),
            full((128, 128)),
            full((1, 128)),
            full((1, 128)),
            full((1, 128)),
            full((128, 1)),
            pl.BlockSpec(memory_space=pltpu.SMEM),
            full((_EMBED_OUT, _EMBED_DIM)),
        ],
        out_specs=pl.BlockSpec((_B,), lambda i: (0,)),
        out_shape=jax.ShapeDtypeStruct((_B,), jnp.float32),
        scratch_shapes=[
            pltpu.VMEM((_B, 128), jnp.float32),
            pltpu.VMEM((_B,), jnp.float32),
        ],
    )(GTlo, GThi, lvalT, W1, b1.reshape(1, 128), g1.reshape(1, 128), be1.reshape(1, 128),
      W2, b2.reshape(1, 128), g2.reshape(1, 128), be2.reshape(1, 128),
      W3, sc, jnp.asarray(_S_MAT_P))


def kernel(x, E, L, bias, W1, b1, g1, be1, W2, b2, g2, be2, W3, b3):
    idxT = x.T.astype(jnp.int32) + jnp.asarray(_OFFS)[:, None]
    d_off = jnp.arange(8, dtype=jnp.int32)[:, None, None]
    v = idxT[None, :, :]
    eidx_h = jnp.where(v < _VMAIN,
                       d_off * _VMAIN + v,
                       _HTBASE + d_off * 64 + (v - _VMAIN))
    eidx_h = eidx_h.reshape(_N_EH)
    lidx = idxT.reshape(_N_L)
    tail_lo = lax.slice(E, (_VMAIN, 0), (_VOCAB, 8)).T.reshape(512)
    tail_hi = lax.slice(E, (_VMAIN, 8), (_VOCAB, _EMBED_DIM)).T.reshape(512)
    tail_l = jnp.concatenate(
        [lax.slice(L, (_VMAIN, 0), (_VOCAB, 1))[:, 0],
         jnp.zeros((64,), jnp.float32)])
    et_lo, l_flat = _compact_half_lo(E.T, L.T, tail_lo, tail_l)
    gt_lo = _build_sc_gather_e(_N_EH)(eidx_h, et_lo)
    et_hi = _compact_half_hi(E.T, tail_hi)
    gt_hi = _build_sc_gather_e(_N_EH)(eidx_h, et_hi)
    lval_flat = _build_sc_gather_l()(lidx, l_flat)
    GTlo = gt_lo.reshape(_HALF, _B // 128, 128)
    GThi = gt_hi.reshape(_HALF, _B // 128, 128)
    lvalT = lval_flat.reshape(_NUM_FIELDS, _B)
    sc = (bias + b3).reshape(1)
    W1p = jnp.take(W1, jnp.asarray(_PERM), axis=0)
    return _tc_mlp(GTlo, GThi, lvalT, W1p, b1, g1, be1, W2, b2, g2, be2, W3, sc)
